# baseline (speedup 1.0000x reference)
import jax
import jax.numpy as jnp
from jax.experimental import pallas as pl
from jax.experimental.pallas import tpu as pltpu

_OFF_HALF = 9
_N_SEG = 4
_TH = 128
_RC = 8
_CC = 256


def _loss_kernel(off_ref, flow_ref, out_ref):
    jt = pl.program_id(1)
    w = off_ref.shape[3]

    acc = None
    for r in range(0, _TH, _RC):
        for c in range(0, w, _CC):
            rs = slice(r, r + _RC)
            cs = slice(c, c + _CC)
            ch = [flow_ref[0, k, rs, cs] for k in range(_N_SEG + 1)]
            seg = []
            for j in range(_N_SEG):
                u = ch[j]
                v = ch[j + 1]
                uu = u * u + v * v
                inv = 1.0 / uu
                wj = u * uu
                lo = jnp.minimum(0.0, wj)
                hi = jnp.maximum(0.0, wj)
                hi = jnp.where(uu > 0.0, hi, -1.0)
                seg.append((u, v, inv, lo, hi))
            msum = None
            for i in range(_OFF_HALF):
                x = off_ref[0, i, rs, cs]
                y = off_ref[0, _OFF_HALF + i, rs, cs]
                xx = x * x
                d1sq = xx + y * y
                cy = [ch[k] * y for k in range(_N_SEG + 1)]
                cx = [ch[k] * x for k in range(1, _N_SEG + 1)]
                msq = None
                any_out = None
                for j, (u, v, inv, lo, hi) in enumerate(seg):
                    s = u * (xx + cy[j + 1])
                    out = (s < lo) | (s > hi)
                    t = cx[j] - cy[j]
                    perpsq = t * t * inv
                    dx = x - u
                    dy = y - v
                    d2sq = dx * dx + dy * dy
                    md = jnp.where(out, d2sq, perpsq)
                    msq = md if msq is None else jnp.minimum(msq, md)
                    any_out = out if any_out is None else any_out | out
                msq = jnp.minimum(msq, jnp.where(any_out, d1sq, jnp.inf))
                m = msq * jax.lax.rsqrt(jnp.maximum(msq, 1e-30))
                msum = m if msum is None else msum + m
            acc = msum if acc is None else acc + msum

    s81 = jnp.sum(acc, axis=-1, keepdims=True)
    s11 = jnp.sum(s81, axis=0, keepdims=True)
    part = jnp.broadcast_to(s11, (8, 128))

    @pl.when((pl.program_id(0) == 0) & (jt == 0))
    def _():
        out_ref[...] = jnp.zeros_like(out_ref)

    out_ref[...] += part


def kernel(offset, optical_flow):
    b, c_off, h, w = offset.shape
    of_num = optical_flow.shape[1] // 2
    ht = h // _TH

    out = pl.pallas_call(
        _loss_kernel,
        out_shape=jax.ShapeDtypeStruct((8, 128), jnp.float32),
        grid=(b, ht),
        in_specs=[
            pl.BlockSpec((1, c_off, _TH, w), lambda i, j: (i, 0, j, 0)),
            pl.BlockSpec((1, of_num + 1, _TH, w), lambda i, j: (i, 0, j, 0)),
        ],
        out_specs=pl.BlockSpec((8, 128), lambda i, j: (0, 0)),
        compiler_params=pltpu.CompilerParams(
            dimension_semantics=("arbitrary", "arbitrary"),
        ),
        name="position_loss_val",
    )(offset, optical_flow)

    return out[0, 0] / (_OFF_HALF * h * w)


# speedup vs baseline: 1.0216x; 1.0216x over previous
import jax
import jax.numpy as jnp
from jax.experimental import pallas as pl
from jax.experimental.pallas import tpu as pltpu

_OFF_HALF = 9
_N_SEG = 4
_TH = 128
_RC = 8
_CC = 128


def _loss_kernel(off_ref, flow_ref, out_ref):
    jt = pl.program_id(1)
    w = off_ref.shape[3]

    acc = None
    for r in range(0, _TH, _RC):
        for c in range(0, w, _CC):
            rs = slice(r, r + _RC)
            cs = slice(c, c + _CC)
            ch = [flow_ref[0, k, rs, cs] for k in range(_N_SEG + 1)]
            seg = []
            for j in range(_N_SEG):
                u = ch[j]
                v = ch[j + 1]
                uu = u * u + v * v
                inv = 1.0 / uu
                wj = u * uu
                lo = jnp.minimum(0.0, wj)
                hi = jnp.maximum(0.0, wj)
                hi = jnp.where(uu > 0.0, hi, -1.0)
                seg.append((u, v, inv, lo, hi))
            msum = None
            for i in range(_OFF_HALF):
                x = off_ref[0, i, rs, cs]
                y = off_ref[0, _OFF_HALF + i, rs, cs]
                xx = x * x
                d1sq = xx + y * y
                cy = [ch[k] * y for k in range(_N_SEG + 1)]
                cx = [ch[k] * x for k in range(1, _N_SEG + 1)]
                msq = None
                any_out = None
                for j, (u, v, inv, lo, hi) in enumerate(seg):
                    s = u * (xx + cy[j + 1])
                    out = (s < lo) | (s > hi)
                    t = cx[j] - cy[j]
                    perpsq = t * t * inv
                    dx = x - u
                    dy = y - v
                    d2sq = dx * dx + dy * dy
                    md = jnp.where(out, d2sq, perpsq)
                    msq = md if msq is None else jnp.minimum(msq, md)
                    any_out = out if any_out is None else any_out | out
                msq = jnp.minimum(msq, jnp.where(any_out, d1sq, jnp.inf))
                m = msq * jax.lax.rsqrt(jnp.maximum(msq, 1e-30))
                msum = m if msum is None else msum + m
            acc = msum if acc is None else acc + msum

    s81 = jnp.sum(acc, axis=-1, keepdims=True)
    s11 = jnp.sum(s81, axis=0, keepdims=True)
    part = jnp.broadcast_to(s11, (8, 128))

    @pl.when((pl.program_id(0) == 0) & (jt == 0))
    def _():
        out_ref[...] = jnp.zeros_like(out_ref)

    out_ref[...] += part


def kernel(offset, optical_flow):
    b, c_off, h, w = offset.shape
    of_num = optical_flow.shape[1] // 2
    ht = h // _TH

    out = pl.pallas_call(
        _loss_kernel,
        out_shape=jax.ShapeDtypeStruct((8, 128), jnp.float32),
        grid=(b, ht),
        in_specs=[
            pl.BlockSpec((1, c_off, _TH, w), lambda i, j: (i, 0, j, 0)),
            pl.BlockSpec((1, of_num + 1, _TH, w), lambda i, j: (i, 0, j, 0)),
        ],
        out_specs=pl.BlockSpec((8, 128), lambda i, j: (0, 0)),
        compiler_params=pltpu.CompilerParams(
            dimension_semantics=("arbitrary", "arbitrary"),
        ),
        name="position_loss_val",
    )(offset, optical_flow)

    return out[0, 0] / (_OFF_HALF * h * w)
